# baseline (device time: 270805 ns/iter reference)
import jax
import jax.numpy as jnp
from jax import lax
from jax.experimental import pallas as pl
from jax.experimental.pallas import tpu as pltpu

N_DEV = 16


def _gelu(y):
    c = 0.7978845608028654
    return 0.5 * y * (1.0 + jnp.tanh(c * (y + 0.044715 * y * y * y)))


def kernel(x, w_mat):
    m, k_per = x.shape
    _, n = w_mat.shape
    chunk = m // N_DEV

    def body(x_ref, w_ref, out_ref, comm_ref, send_sems, recv_sems, credit_sem):
        my = lax.axis_index("i")
        left = lax.rem(my - 1 + N_DEV, N_DEV)
        right = lax.rem(my + 1, N_DEV)

        out_ref[...] = jnp.dot(
            x_ref[...], w_ref[...], preferred_element_type=jnp.float32
        )

        barrier = pltpu.get_barrier_semaphore()
        pl.semaphore_signal(
            barrier, inc=1, device_id=(left,), device_id_type=pl.DeviceIdType.MESH
        )
        pl.semaphore_signal(
            barrier, inc=1, device_id=(right,), device_id_type=pl.DeviceIdType.MESH
        )
        pl.semaphore_wait(barrier, 2)

        comm_ref[0, :, :] = out_ref[pl.ds(my * chunk, chunk), :].astype(jnp.bfloat16)

        n_steps = 2 * (N_DEV - 1)
        for s in range(n_steps):
            send_slot = s % 2
            recv_slot = (s + 1) % 2
            rdma = pltpu.make_async_remote_copy(
                src_ref=comm_ref.at[send_slot],
                dst_ref=comm_ref.at[recv_slot],
                send_sem=send_sems.at[send_slot],
                recv_sem=recv_sems.at[recv_slot],
                device_id=(right,),
                device_id_type=pl.DeviceIdType.MESH,
            )
            if s >= 1:
                pl.semaphore_wait(credit_sem, 1)
            rdma.start()
            rdma.wait_send()
            if s < n_steps - 1:
                pl.semaphore_signal(
                    credit_sem,
                    inc=1,
                    device_id=(left,),
                    device_id_type=pl.DeviceIdType.MESH,
                )
            rdma.wait_recv()

            if s < N_DEV - 2:
                c = lax.rem(my - s - 1 + 2 * N_DEV, N_DEV)
                acc = out_ref[pl.ds(c * chunk, chunk), :]
                comm_ref[recv_slot, :, :] = (
                    comm_ref[recv_slot, :, :].astype(jnp.float32) + acc
                ).astype(jnp.bfloat16)
            elif s == N_DEV - 2:
                c = lax.rem(my + 1, N_DEV)
                total = comm_ref[recv_slot, :, :].astype(jnp.float32) + out_ref[
                    pl.ds(c * chunk, chunk), :
                ]
                g = _gelu(total)
                out_ref[pl.ds(c * chunk, chunk), :] = g
                comm_ref[recv_slot, :, :] = g.astype(jnp.bfloat16)
            else:
                t = s - (N_DEV - 1)
                o = lax.rem(my - t + 2 * N_DEV, N_DEV)
                out_ref[pl.ds(o * chunk, chunk), :] = comm_ref[
                    recv_slot, :, :
                ].astype(jnp.float32)

    return pl.pallas_call(
        body,
        out_shape=jax.ShapeDtypeStruct((m, n), jnp.float32),
        in_specs=[
            pl.BlockSpec(memory_space=pltpu.VMEM),
            pl.BlockSpec(memory_space=pltpu.VMEM),
        ],
        out_specs=pl.BlockSpec(memory_space=pltpu.VMEM),
        scratch_shapes=[
            pltpu.VMEM((2, chunk, n), jnp.bfloat16),
            pltpu.SemaphoreType.DMA((2,)),
            pltpu.SemaphoreType.DMA((2,)),
            pltpu.SemaphoreType.REGULAR,
        ],
        compiler_params=pltpu.CompilerParams(collective_id=0),
    )(x, w_mat)


# device time: 157719 ns/iter; 1.7170x vs baseline; 1.7170x over previous
import jax
import jax.numpy as jnp
from jax import lax
from jax.experimental import pallas as pl
from jax.experimental.pallas import tpu as pltpu

N_DEV = 16
MESH = pl.DeviceIdType.MESH


def _gelu(y):
    c = 0.7978845608028654
    return 0.5 * y * (1.0 + jnp.tanh(c * (y + 0.044715 * y * y * y)))


def kernel(x, w_mat):
    m, k_per = x.shape
    _, n = w_mat.shape
    chunk = m // N_DEV
    bf16 = jnp.bfloat16
    f32 = jnp.float32

    R_LAST = 15
    L_LAST = 13

    def body(x_ref, w_ref, out_ref, comm_r, comm_l,
             ssem_r, rsem_r, ssem_l, rsem_l, cred_r, cred_l):
        my = lax.axis_index("i")
        left = lax.rem(my - 1 + N_DEV, N_DEV)
        right = lax.rem(my + 1, N_DEV)

        out_ref[...] = jnp.dot(
            x_ref[...], w_ref[...], preferred_element_type=f32
        )

        barrier = pltpu.get_barrier_semaphore()
        for nbr in (left, right):
            pl.semaphore_signal(barrier, inc=1, device_id=(nbr,),
                                device_id_type=MESH)
        pl.semaphore_wait(barrier, 2)

        def row(off):
            return lax.rem(my + off + 2 * N_DEV, N_DEV) * chunk

        def acc(off):
            return out_ref[pl.ds(row(off), chunk), :]

        def mk(comm, ssem, rsem, step, dst):
            return pltpu.make_async_remote_copy(
                src_ref=comm.at[step % 2],
                dst_ref=comm.at[(step + 1) % 2],
                send_sem=ssem.at[step % 2],
                recv_sem=rsem.at[(step + 1) % 2],
                device_id=(dst,),
                device_id_type=MESH,
            )

        comm_r[0, :, :] = acc(8).astype(bf16)
        comm_l[0, :, :] = acc(-7).astype(bf16)

        for s in range(16):
            l_step = s - 1
            l_on = 0 <= l_step <= L_LAST

            r = mk(comm_r, ssem_r, rsem_r, s, right)
            l = mk(comm_l, ssem_l, rsem_l, l_step, left) if l_on else None

            if s >= 1:
                pl.semaphore_wait(cred_r, 1)
            if l_on and l_step >= 1:
                pl.semaphore_wait(cred_l, 1)
            r.start()
            if l_on:
                l.start()
            r.wait_send()
            if s < R_LAST:
                pl.semaphore_signal(cred_r, inc=1, device_id=(left,),
                                    device_id_type=MESH)
            if l_on:
                l.wait_send()
                if l_step < L_LAST:
                    pl.semaphore_signal(cred_l, inc=1, device_id=(right,),
                                        device_id_type=MESH)
            r.wait_recv()
            if l_on:
                l.wait_recv()

            r_rs = (s + 1) % 2
            l_rs = (l_step + 1) % 2

            if s <= 6:
                comm_r[r_rs, :, :] = (
                    comm_r[r_rs, :, :].astype(f32) + acc(7 - s)
                ).astype(bf16)
                if l_on:
                    comm_l[l_rs, :, :] = (
                        comm_l[l_rs, :, :].astype(f32) + acc(l_step - 6)
                    ).astype(bf16)
            elif s == 7:
                total = (comm_r[0, :, :].astype(f32)
                         + comm_l[1, :, :].astype(f32) + acc(0))
                g = _gelu(total)
                out_ref[pl.ds(row(0), chunk), :] = g
                gb = g.astype(bf16)
                comm_r[0, :, :] = gb
                comm_l[1, :, :] = gb
            else:
                out_ref[pl.ds(row(7 - s), chunk), :] = (
                    comm_r[r_rs, :, :].astype(f32)
                )
                if l_on:
                    out_ref[pl.ds(row(s - 7), chunk), :] = (
                        comm_l[l_rs, :, :].astype(f32)
                    )

    return pl.pallas_call(
        body,
        out_shape=jax.ShapeDtypeStruct((m, n), f32),
        in_specs=[
            pl.BlockSpec(memory_space=pltpu.VMEM),
            pl.BlockSpec(memory_space=pltpu.VMEM),
        ],
        out_specs=pl.BlockSpec(memory_space=pltpu.VMEM),
        scratch_shapes=[
            pltpu.VMEM((2, chunk, n), bf16),
            pltpu.VMEM((2, chunk, n), bf16),
            pltpu.SemaphoreType.DMA((2,)),
            pltpu.SemaphoreType.DMA((2,)),
            pltpu.SemaphoreType.DMA((2,)),
            pltpu.SemaphoreType.DMA((2,)),
            pltpu.SemaphoreType.REGULAR,
            pltpu.SemaphoreType.REGULAR,
        ],
        compiler_params=pltpu.CompilerParams(collective_id=0),
    )(x, w_mat)


# device time: 114587 ns/iter; 2.3633x vs baseline; 1.3764x over previous
import jax
import jax.numpy as jnp
from jax import lax
from jax.experimental import pallas as pl
from jax.experimental.pallas import tpu as pltpu

N_DEV = 16
MESH = pl.DeviceIdType.MESH


def _gelu(y):
    c = 0.7978845608028654
    return 0.5 * y * (1.0 + jnp.tanh(c * (y + 0.044715 * y * y * y)))


def kernel(x, w_mat):
    m, k_per = x.shape
    _, n = w_mat.shape
    chunk = m // N_DEV
    half = n // 2
    bf16 = jnp.bfloat16
    f32 = jnp.float32

    R_LAST = 15
    L_LAST = 13

    def body(x_ref, w_ref, out_ref,
             comm_ra, comm_rb, comm_la, comm_lb,
             ssem_ra, rsem_ra, ssem_rb, rsem_rb,
             ssem_la, rsem_la, ssem_lb, rsem_lb,
             cred_ra, cred_rb, cred_la, cred_lb):
        my = lax.axis_index("i")
        left = lax.rem(my - 1 + N_DEV, N_DEV)
        right = lax.rem(my + 1, N_DEV)

        out_ref[...] = jnp.dot(
            x_ref[...], w_ref[...], preferred_element_type=f32
        )

        barrier = pltpu.get_barrier_semaphore()
        for nbr in (left, right):
            pl.semaphore_signal(barrier, inc=1, device_id=(nbr,),
                                device_id_type=MESH)
        pl.semaphore_wait(barrier, 2)

        def row(off):
            return lax.rem(my + off + 2 * N_DEV, N_DEV) * chunk

        def acc_h(off, co):
            return out_ref[pl.ds(row(off), chunk), co:co + half]

        class Lane:
            def __init__(self, comm, ssem, rsem, cred, dst, cred_to, co, last):
                self.comm, self.ssem, self.rsem, self.cred = comm, ssem, rsem, cred
                self.dst, self.cred_to, self.co, self.last = dst, cred_to, co, last

            def mk(self, k):
                return pltpu.make_async_remote_copy(
                    src_ref=self.comm.at[k % 2],
                    dst_ref=self.comm.at[(k + 1) % 2],
                    send_sem=self.ssem.at[k % 2],
                    recv_sem=self.rsem.at[(k + 1) % 2],
                    device_id=(self.dst,),
                    device_id_type=MESH,
                )

            def issue(self, k):
                if k >= 1:
                    pl.semaphore_wait(self.cred, 1)
                self.mk(k).start()

            def finish_send(self, k):
                self.mk(k).wait_send()
                if k < self.last:
                    pl.semaphore_signal(self.cred, inc=1,
                                        device_id=(self.cred_to,),
                                        device_id_type=MESH)

        ra = Lane(comm_ra, ssem_ra, rsem_ra, cred_ra, right, left, 0, R_LAST)
        rb = Lane(comm_rb, ssem_rb, rsem_rb, cred_rb, right, left, half, R_LAST)
        la = Lane(comm_la, ssem_la, rsem_la, cred_la, left, right, 0, L_LAST)
        lb = Lane(comm_lb, ssem_lb, rsem_lb, cred_lb, left, right, half, L_LAST)

        def process_r(lane, k):
            lane.mk(k).wait_recv()
            rs = (k + 1) % 2
            if k <= 6:
                lane.comm[rs, :, :] = (
                    lane.comm[rs, :, :].astype(f32) + acc_h(7 - k, lane.co)
                ).astype(bf16)
            elif k == 7:
                pass
            else:
                out_ref[pl.ds(row(7 - k), chunk), lane.co:lane.co + half] = (
                    lane.comm[rs, :, :].astype(f32)
                )

        def process_l(lane, k):
            lane.mk(k).wait_recv()
            rs = (k + 1) % 2
            if k <= 5:
                lane.comm[rs, :, :] = (
                    lane.comm[rs, :, :].astype(f32) + acc_h(k - 6, lane.co)
                ).astype(bf16)
            elif k == 6:
                pass
            else:
                out_ref[pl.ds(row(k - 6), chunk), lane.co:lane.co + half] = (
                    lane.comm[rs, :, :].astype(f32)
                )

        def combine(r_lane, l_lane):
            co = r_lane.co
            total = (r_lane.comm[0, :, :].astype(f32)
                     + l_lane.comm[1, :, :].astype(f32) + acc_h(0, co))
            g = _gelu(total)
            out_ref[pl.ds(row(0), chunk), co:co + half] = g
            gb = g.astype(bf16)
            r_lane.comm[0, :, :] = gb
            l_lane.comm[1, :, :] = gb

        comm_ra[0, :, :] = acc_h(8, 0).astype(bf16)
        comm_rb[0, :, :] = acc_h(8, half).astype(bf16)
        comm_la[0, :, :] = acc_h(-7, 0).astype(bf16)
        comm_lb[0, :, :] = acc_h(-7, half).astype(bf16)

        for s in range(17):
            if s <= R_LAST:
                ra.issue(s)
            if 1 <= s <= L_LAST + 1:
                la.issue(s - 1)
            if 1 <= s <= R_LAST + 1:
                rb.finish_send(s - 1)
                process_r(rb, s - 1)
            if 2 <= s <= L_LAST + 2:
                lb.finish_send(s - 2)
                process_l(lb, s - 2)
            if s == 8:
                combine(rb, lb)
            if s <= R_LAST:
                rb.issue(s)
            if 1 <= s <= L_LAST + 1:
                lb.issue(s - 1)
            if s <= R_LAST:
                ra.finish_send(s)
                process_r(ra, s)
            if 1 <= s <= L_LAST + 1:
                la.finish_send(s - 1)
                process_l(la, s - 1)
            if s == 7:
                combine(ra, la)

    dma2 = pltpu.SemaphoreType.DMA((2,))
    reg = pltpu.SemaphoreType.REGULAR
    return pl.pallas_call(
        body,
        out_shape=jax.ShapeDtypeStruct((m, n), f32),
        in_specs=[
            pl.BlockSpec(memory_space=pltpu.VMEM),
            pl.BlockSpec(memory_space=pltpu.VMEM),
        ],
        out_specs=pl.BlockSpec(memory_space=pltpu.VMEM),
        scratch_shapes=[
            pltpu.VMEM((2, chunk, half), bf16),
            pltpu.VMEM((2, chunk, half), bf16),
            pltpu.VMEM((2, chunk, half), bf16),
            pltpu.VMEM((2, chunk, half), bf16),
            dma2, dma2, dma2, dma2,
            dma2, dma2, dma2, dma2,
            reg, reg, reg, reg,
        ],
        compiler_params=pltpu.CompilerParams(collective_id=0),
    )(x, w_mat)


# device time: 113789 ns/iter; 2.3799x vs baseline; 1.0070x over previous
import jax
import jax.numpy as jnp
from jax import lax
from jax.experimental import pallas as pl
from jax.experimental.pallas import tpu as pltpu

N_DEV = 16
MESH = pl.DeviceIdType.MESH


def _gelu(y):
    c = 0.7978845608028654
    return 0.5 * y * (1.0 + jnp.tanh(c * (y + 0.044715 * y * y * y)))


def kernel(x, w_mat):
    m, k_per = x.shape
    _, n = w_mat.shape
    chunk = m // N_DEV
    half = n // 2
    bf16 = jnp.bfloat16
    f32 = jnp.float32

    R_LAST = 15
    L_LAST = 13

    def body(x_ref, w_ref, out_ref,
             comm_ra, comm_rb, comm_la, comm_lb,
             ssem_ra, rsem_ra, ssem_rb, rsem_rb,
             ssem_la, rsem_la, ssem_lb, rsem_lb,
             cred_ra, cred_rb, cred_la, cred_lb):
        my = lax.axis_index("i")
        left = lax.rem(my - 1 + N_DEV, N_DEV)
        right = lax.rem(my + 1, N_DEV)

        out_ref[...] = jnp.dot(
            x_ref[...], w_ref[...], preferred_element_type=f32
        )

        barrier = pltpu.get_barrier_semaphore()
        for nbr in (left, right):
            pl.semaphore_signal(barrier, inc=1, device_id=(nbr,),
                                device_id_type=MESH)
        pl.semaphore_wait(barrier, 2)

        def row(off):
            return lax.rem(my + off + 2 * N_DEV, N_DEV) * chunk

        def acc_h(off, co):
            return out_ref[pl.ds(row(off), chunk), co:co + half]

        class Lane:
            def __init__(self, comm, ssem, rsem, cred, dst, cred_to, co, last):
                self.comm, self.ssem, self.rsem, self.cred = comm, ssem, rsem, cred
                self.dst, self.cred_to, self.co, self.last = dst, cred_to, co, last

            def mk(self, k):
                return pltpu.make_async_remote_copy(
                    src_ref=self.comm.at[k % 2],
                    dst_ref=self.comm.at[(k + 1) % 2],
                    send_sem=self.ssem.at[k % 2],
                    recv_sem=self.rsem.at[(k + 1) % 2],
                    device_id=(self.dst,),
                    device_id_type=MESH,
                )

            def issue(self, k):
                if k >= 1:
                    pl.semaphore_wait(self.cred, 1)
                self.mk(k).start()

            def finish_send(self, k):
                self.mk(k).wait_send()
                if k < self.last:
                    pl.semaphore_signal(self.cred, inc=1,
                                        device_id=(self.cred_to,),
                                        device_id_type=MESH)

        ra = Lane(comm_ra, ssem_ra, rsem_ra, cred_ra, right, left, 0, R_LAST)
        rb = Lane(comm_rb, ssem_rb, rsem_rb, cred_rb, right, left, half, R_LAST)
        la = Lane(comm_la, ssem_la, rsem_la, cred_la, left, right, 0, L_LAST)
        lb = Lane(comm_lb, ssem_lb, rsem_lb, cred_lb, left, right, half, L_LAST)

        def add_r(lane, k):
            rs = (k + 1) % 2
            lane.comm[rs, :, :] = (
                lane.comm[rs, :, :].astype(f32) + acc_h(7 - k, lane.co)
            ).astype(bf16)

        def add_l(lane, k):
            rs = (k + 1) % 2
            lane.comm[rs, :, :] = (
                lane.comm[rs, :, :].astype(f32) + acc_h(k - 6, lane.co)
            ).astype(bf16)

        def store_r(lane, k):
            rs = (k + 1) % 2
            out_ref[pl.ds(row(7 - k), chunk), lane.co:lane.co + half] = (
                lane.comm[rs, :, :].astype(f32)
            )

        def store_l(lane, k):
            rs = (k + 1) % 2
            out_ref[pl.ds(row(k - 6), chunk), lane.co:lane.co + half] = (
                lane.comm[rs, :, :].astype(f32)
            )

        def combine(r_lane, l_lane):
            co = r_lane.co
            total = (r_lane.comm[0, :, :].astype(f32)
                     + l_lane.comm[1, :, :].astype(f32) + acc_h(0, co))
            g = _gelu(total)
            out_ref[pl.ds(row(0), chunk), co:co + half] = g
            gb = g.astype(bf16)
            r_lane.comm[0, :, :] = gb
            l_lane.comm[1, :, :] = gb

        comm_ra[0, :, :] = acc_h(8, 0).astype(bf16)
        comm_rb[0, :, :] = acc_h(8, half).astype(bf16)
        comm_la[0, :, :] = acc_h(-7, 0).astype(bf16)
        comm_lb[0, :, :] = acc_h(-7, half).astype(bf16)

        pend_ra = pend_la = None
        for s in range(17):
            if s <= R_LAST:
                ra.issue(s)
            if 1 <= s <= L_LAST + 1:
                la.issue(s - 1)
            if pend_ra is not None:
                store_r(ra, pend_ra)
                pend_ra = None
            if pend_la is not None:
                store_l(la, pend_la)
                pend_la = None
            if 1 <= s <= R_LAST + 1:
                rb.finish_send(s - 1)
                rb.mk(s - 1).wait_recv()
                if s - 1 <= 6:
                    add_r(rb, s - 1)
            if 2 <= s <= L_LAST + 2:
                lb.finish_send(s - 2)
                lb.mk(s - 2).wait_recv()
                if s - 2 <= 5:
                    add_l(lb, s - 2)
            if s == 8:
                combine(rb, lb)
            if s <= R_LAST:
                rb.issue(s)
            if 1 <= s <= L_LAST + 1:
                lb.issue(s - 1)
            if 1 <= s <= R_LAST + 1 and s - 1 >= 8:
                store_r(rb, s - 1)
            if 2 <= s <= L_LAST + 2 and s - 2 >= 7:
                store_l(lb, s - 2)
            if s <= R_LAST:
                ra.finish_send(s)
                ra.mk(s).wait_recv()
                if s <= 6:
                    add_r(ra, s)
                elif s >= 8:
                    pend_ra = s
            if 1 <= s <= L_LAST + 1:
                la.finish_send(s - 1)
                la.mk(s - 1).wait_recv()
                if s - 1 <= 5:
                    add_l(la, s - 1)
                elif s - 1 >= 7:
                    pend_la = s - 1
            if s == 7:
                combine(ra, la)

    dma2 = pltpu.SemaphoreType.DMA((2,))
    reg = pltpu.SemaphoreType.REGULAR
    return pl.pallas_call(
        body,
        out_shape=jax.ShapeDtypeStruct((m, n), f32),
        in_specs=[
            pl.BlockSpec(memory_space=pltpu.VMEM),
            pl.BlockSpec(memory_space=pltpu.VMEM),
        ],
        out_specs=pl.BlockSpec(memory_space=pltpu.VMEM),
        scratch_shapes=[
            pltpu.VMEM((2, chunk, half), bf16),
            pltpu.VMEM((2, chunk, half), bf16),
            pltpu.VMEM((2, chunk, half), bf16),
            pltpu.VMEM((2, chunk, half), bf16),
            dma2, dma2, dma2, dma2,
            dma2, dma2, dma2, dma2,
            reg, reg, reg, reg,
        ],
        compiler_params=pltpu.CompilerParams(collective_id=0),
    )(x, w_mat)


# device time: 108152 ns/iter; 2.5039x vs baseline; 1.0521x over previous
import jax
import jax.numpy as jnp
from jax import lax
from jax.experimental import pallas as pl
from jax.experimental.pallas import tpu as pltpu

N_DEV = 16
MESH = pl.DeviceIdType.MESH
H = 4
DEPTH = 3

R_LAST = 15
L_LAST = 13


def _gelu(y):
    c = 0.7978845608028654
    return 0.5 * y * (1.0 + jnp.tanh(c * (y + 0.044715 * y * y * y)))


def kernel(x, w_mat):
    m, k_per = x.shape
    _, n = w_mat.shape
    chunk = m // N_DEV
    cw = n // H
    bf16 = jnp.bfloat16
    f32 = jnp.float32

    def body(x_ref, w_ref, out_ref, *scratch):
        comms = scratch[0:2 * H]
        ssems = scratch[2 * H:4 * H]
        rsems = scratch[4 * H:6 * H]
        creds = scratch[6 * H:8 * H]

        my = lax.axis_index("i")
        left = lax.rem(my - 1 + N_DEV, N_DEV)
        right = lax.rem(my + 1, N_DEV)

        def row(off):
            return lax.rem(my + off + 2 * N_DEV, N_DEV) * chunk

        def matmul(off):
            r0 = row(off)
            out_ref[pl.ds(r0, chunk), :] = jnp.dot(
                x_ref[pl.ds(r0, chunk), :], w_ref[...],
                preferred_element_type=f32,
            )

        def acc_q(off, co):
            return out_ref[pl.ds(row(off), chunk), co:co + cw]

        class Lane:
            def __init__(self, i, dst, cred_to, co, last):
                self.comm, self.ssem = comms[i], ssems[i]
                self.rsem, self.cred = rsems[i], creds[i]
                self.dst, self.cred_to, self.co, self.last = dst, cred_to, co, last

            def mk(self, k):
                return pltpu.make_async_remote_copy(
                    src_ref=self.comm.at[k % DEPTH],
                    dst_ref=self.comm.at[(k + 1) % DEPTH],
                    send_sem=self.ssem.at[k % DEPTH],
                    recv_sem=self.rsem.at[(k + 1) % DEPTH],
                    device_id=(self.dst,),
                    device_id_type=MESH,
                )

            def issue(self, k):
                if k >= DEPTH - 1:
                    pl.semaphore_wait(self.cred, 1)
                self.mk(k).start()

            def finish_send(self, k):
                self.mk(k).wait_send()
                if k <= self.last - (DEPTH - 1):
                    pl.semaphore_signal(self.cred, inc=1,
                                        device_id=(self.cred_to,),
                                        device_id_type=MESH)

        r_lanes = [Lane(j, right, left, j * cw, R_LAST) for j in range(H)]
        l_lanes = [Lane(H + j, left, right, j * cw, L_LAST) for j in range(H)]

        def add_r(lane, k):
            rs = (k + 1) % DEPTH
            lane.comm[rs, :, :] = (
                lane.comm[rs, :, :].astype(f32) + acc_q(7 - k, lane.co)
            ).astype(bf16)

        def add_l(lane, k):
            rs = (k + 1) % DEPTH
            lane.comm[rs, :, :] = (
                lane.comm[rs, :, :].astype(f32) + acc_q(k - 6, lane.co)
            ).astype(bf16)

        def store_r(lane, k):
            rs = (k + 1) % DEPTH
            out_ref[pl.ds(row(7 - k), chunk), lane.co:lane.co + cw] = (
                lane.comm[rs, :, :].astype(f32)
            )

        def store_l(lane, k):
            rs = (k + 1) % DEPTH
            out_ref[pl.ds(row(k - 6), chunk), lane.co:lane.co + cw] = (
                lane.comm[rs, :, :].astype(f32)
            )

        def combine(r_lane, l_lane):
            r_fin = (7 + 1) % DEPTH
            l_fin = (6 + 1) % DEPTH
            co = r_lane.co
            total = (r_lane.comm[r_fin, :, :].astype(f32)
                     + l_lane.comm[l_fin, :, :].astype(f32) + acc_q(0, co))
            g = _gelu(total)
            out_ref[pl.ds(row(0), chunk), co:co + cw] = g
            gb = g.astype(bf16)
            r_lane.comm[r_fin, :, :] = gb
            l_lane.comm[l_fin, :, :] = gb

        matmul(8)
        matmul(-7)

        barrier = pltpu.get_barrier_semaphore()
        for nbr in (left, right):
            pl.semaphore_signal(barrier, inc=1, device_id=(nbr,),
                                device_id_type=MESH)
        pl.semaphore_wait(barrier, 2)

        for lane in r_lanes:
            lane.comm[0, :, :] = acc_q(8, lane.co).astype(bf16)
        for lane in l_lanes:
            lane.comm[0, :, :] = acc_q(-7, lane.co).astype(bf16)

        pend_r = pend_l = None
        for s in range(17):
            ra, la = r_lanes[0], l_lanes[0]
            if s <= R_LAST:
                ra.issue(s)
            if 1 <= s <= L_LAST + 1:
                la.issue(s - 1)
            if s == 0:
                for off in (7, -6, 6, -5, 5, -4, 4, -3, 3, -2, 2, -1, 1, 0):
                    matmul(off)
            if pend_r is not None:
                store_r(ra, pend_r)
                pend_r = None
            if pend_l is not None:
                store_l(la, pend_l)
                pend_l = None
            for j in range(1, H):
                rj, lj = r_lanes[j], l_lanes[j]
                if 1 <= s <= R_LAST + 1:
                    rj.finish_send(s - 1)
                    rj.mk(s - 1).wait_recv()
                    if s - 1 <= 6:
                        add_r(rj, s - 1)
                if 2 <= s <= L_LAST + 2:
                    lj.finish_send(s - 2)
                    lj.mk(s - 2).wait_recv()
                    if s - 2 <= 5:
                        add_l(lj, s - 2)
                if s == 8:
                    combine(rj, lj)
                if s <= R_LAST:
                    rj.issue(s)
                if 1 <= s <= L_LAST + 1:
                    lj.issue(s - 1)
                if 1 <= s <= R_LAST + 1 and s - 1 >= 8:
                    store_r(rj, s - 1)
                if 2 <= s <= L_LAST + 2 and s - 2 >= 7:
                    store_l(lj, s - 2)
            if s <= R_LAST:
                ra.finish_send(s)
                ra.mk(s).wait_recv()
                if s <= 6:
                    add_r(ra, s)
                elif s >= 8:
                    pend_r = s
            if 1 <= s <= L_LAST + 1:
                la.finish_send(s - 1)
                la.mk(s - 1).wait_recv()
                if s - 1 <= 5:
                    add_l(la, s - 1)
                elif s - 1 >= 7:
                    pend_l = s - 1
            if s == 7:
                combine(ra, la)

    comm_shape = pltpu.VMEM((DEPTH, chunk, cw), bf16)
    dma = pltpu.SemaphoreType.DMA((DEPTH,))
    reg = pltpu.SemaphoreType.REGULAR
    return pl.pallas_call(
        body,
        out_shape=jax.ShapeDtypeStruct((m, n), f32),
        in_specs=[
            pl.BlockSpec(memory_space=pltpu.VMEM),
            pl.BlockSpec(memory_space=pltpu.VMEM),
        ],
        out_specs=pl.BlockSpec(memory_space=pltpu.VMEM),
        scratch_shapes=(
            [comm_shape] * (2 * H)
            + [dma] * (2 * H)
            + [dma] * (2 * H)
            + [reg] * (2 * H)
        ),
        compiler_params=pltpu.CompilerParams(collective_id=0),
    )(x, w_mat)


# device time: 108082 ns/iter; 2.5056x vs baseline; 1.0006x over previous
import jax
import jax.numpy as jnp
from jax import lax
from jax.experimental import pallas as pl
from jax.experimental.pallas import tpu as pltpu

N_DEV = 16
MESH = pl.DeviceIdType.MESH
H = 4
DEPTH = 3

R_LAST = 15
L_LAST = 13


def _gelu(y):
    c = 0.7978845608028654
    return 0.5 * y * (1.0 + jnp.tanh(c * (y + 0.044715 * y * y * y)))


def kernel(x, w_mat):
    m, k_per = x.shape
    _, n = w_mat.shape
    chunk = m // N_DEV
    cw = n // H
    bf16 = jnp.bfloat16
    f32 = jnp.float32

    def body(x_ref, w_ref, out_ref, *scratch):
        comms = scratch[0:2 * H]
        ssems = scratch[2 * H:4 * H]
        rsems = scratch[4 * H:6 * H]
        creds = scratch[6 * H:8 * H]

        my = lax.axis_index("i")
        left = lax.rem(my - 1 + N_DEV, N_DEV)
        right = lax.rem(my + 1, N_DEV)

        def row(off):
            return lax.rem(my + off + 2 * N_DEV, N_DEV) * chunk

        def matmul(off):
            r0 = row(off)
            out_ref[pl.ds(r0, chunk), :] = jnp.dot(
                x_ref[pl.ds(r0, chunk), :], w_ref[...],
                preferred_element_type=f32,
            )

        def acc_q(off, co):
            return out_ref[pl.ds(row(off), chunk), co:co + cw]

        class Lane:
            def __init__(self, i, dst, cred_to, co, last):
                self.comm, self.ssem = comms[i], ssems[i]
                self.rsem, self.cred = rsems[i], creds[i]
                self.dst, self.cred_to, self.co, self.last = dst, cred_to, co, last

            def mk(self, k):
                return pltpu.make_async_remote_copy(
                    src_ref=self.comm.at[k % DEPTH],
                    dst_ref=self.comm.at[(k + 1) % DEPTH],
                    send_sem=self.ssem.at[k % DEPTH],
                    recv_sem=self.rsem.at[(k + 1) % DEPTH],
                    device_id=(self.dst,),
                    device_id_type=MESH,
                )

            def issue(self, k):
                if k >= DEPTH - 1:
                    pl.semaphore_wait(self.cred, 1)
                self.mk(k).start()

            def finish_send(self, k):
                self.mk(k).wait_send()
                if k <= self.last - (DEPTH - 1):
                    pl.semaphore_signal(self.cred, inc=1,
                                        device_id=(self.cred_to,),
                                        device_id_type=MESH)

        r_lanes = [Lane(j, right, left, j * cw, R_LAST) for j in range(H)]
        l_lanes = [Lane(H + j, left, right, j * cw, L_LAST) for j in range(H)]

        def add_r(lane, k):
            rs = (k + 1) % DEPTH
            lane.comm[rs, :, :] = (
                lane.comm[rs, :, :] + acc_q(7 - k, lane.co).astype(bf16)
            )

        def add_l(lane, k):
            rs = (k + 1) % DEPTH
            lane.comm[rs, :, :] = (
                lane.comm[rs, :, :] + acc_q(k - 6, lane.co).astype(bf16)
            )

        def store_r(lane, k):
            rs = (k + 1) % DEPTH
            out_ref[pl.ds(row(7 - k), chunk), lane.co:lane.co + cw] = (
                lane.comm[rs, :, :].astype(f32)
            )

        def store_l(lane, k):
            rs = (k + 1) % DEPTH
            out_ref[pl.ds(row(k - 6), chunk), lane.co:lane.co + cw] = (
                lane.comm[rs, :, :].astype(f32)
            )

        def combine(r_lane, l_lane):
            r_fin = (7 + 1) % DEPTH
            l_fin = (6 + 1) % DEPTH
            co = r_lane.co
            total = (r_lane.comm[r_fin, :, :].astype(f32)
                     + l_lane.comm[l_fin, :, :].astype(f32) + acc_q(0, co))
            g = _gelu(total)
            out_ref[pl.ds(row(0), chunk), co:co + cw] = g
            gb = g.astype(bf16)
            r_lane.comm[r_fin, :, :] = gb
            l_lane.comm[l_fin, :, :] = gb

        matmul(8)
        matmul(-7)

        barrier = pltpu.get_barrier_semaphore()
        for nbr in (left, right):
            pl.semaphore_signal(barrier, inc=1, device_id=(nbr,),
                                device_id_type=MESH)
        pl.semaphore_wait(barrier, 2)

        for lane in r_lanes:
            lane.comm[0, :, :] = acc_q(8, lane.co).astype(bf16)
        for lane in l_lanes:
            lane.comm[0, :, :] = acc_q(-7, lane.co).astype(bf16)

        pend_r = pend_l = None
        for s in range(17):
            ra, la = r_lanes[0], l_lanes[0]
            if s <= R_LAST:
                ra.issue(s)
            if 1 <= s <= L_LAST + 1:
                la.issue(s - 1)
            if s == 0:
                for off in (7, -6, 6, -5, 5, -4, 4, -3, 3, -2, 2, -1, 1, 0):
                    matmul(off)
            if pend_r is not None:
                store_r(ra, pend_r)
                pend_r = None
            if pend_l is not None:
                store_l(la, pend_l)
                pend_l = None
            for j in range(1, H):
                rj, lj = r_lanes[j], l_lanes[j]
                if s == 8:
                    rj.finish_send(7)
                    rj.mk(7).wait_recv()
                    lj.finish_send(6)
                    lj.mk(6).wait_recv()
                    combine(rj, lj)
                    rj.issue(8)
                    lj.issue(7)
                else:
                    if 1 <= s <= R_LAST + 1:
                        rj.finish_send(s - 1)
                        rj.mk(s - 1).wait_recv()
                        if s - 1 <= 6:
                            add_r(rj, s - 1)
                    if s <= R_LAST:
                        rj.issue(s)
                    if 2 <= s <= L_LAST + 2:
                        lj.finish_send(s - 2)
                        lj.mk(s - 2).wait_recv()
                        if s - 2 <= 5:
                            add_l(lj, s - 2)
                    if 1 <= s <= L_LAST + 1:
                        lj.issue(s - 1)
                if 1 <= s <= R_LAST + 1 and s - 1 >= 8:
                    store_r(rj, s - 1)
                if 2 <= s <= L_LAST + 2 and s - 2 >= 7:
                    store_l(lj, s - 2)
            if s <= R_LAST:
                ra.finish_send(s)
                ra.mk(s).wait_recv()
                if s <= 6:
                    add_r(ra, s)
                elif s >= 8:
                    pend_r = s
            if 1 <= s <= L_LAST + 1:
                la.finish_send(s - 1)
                la.mk(s - 1).wait_recv()
                if s - 1 <= 5:
                    add_l(la, s - 1)
                elif s - 1 >= 7:
                    pend_l = s - 1
            if s == 7:
                combine(ra, la)

    comm_shape = pltpu.VMEM((DEPTH, chunk, cw), bf16)
    dma = pltpu.SemaphoreType.DMA((DEPTH,))
    reg = pltpu.SemaphoreType.REGULAR
    return pl.pallas_call(
        body,
        out_shape=jax.ShapeDtypeStruct((m, n), f32),
        in_specs=[
            pl.BlockSpec(memory_space=pltpu.VMEM),
            pl.BlockSpec(memory_space=pltpu.VMEM),
        ],
        out_specs=pl.BlockSpec(memory_space=pltpu.VMEM),
        scratch_shapes=(
            [comm_shape] * (2 * H)
            + [dma] * (2 * H)
            + [dma] * (2 * H)
            + [reg] * (2 * H)
        ),
        compiler_params=pltpu.CompilerParams(collective_id=0),
    )(x, w_mat)


# device time: 105154 ns/iter; 2.5753x vs baseline; 1.0278x over previous
import jax
import jax.numpy as jnp
from jax import lax
from jax.experimental import pallas as pl
from jax.experimental.pallas import tpu as pltpu

N_DEV = 16
MESH = pl.DeviceIdType.MESH
H = 4
DEPTH = 3

R_LAST = 15
L_LAST = 13


def _gelu(y):
    c = 0.7978845608028654
    return 0.5 * y * (1.0 + jnp.tanh(c * (y + 0.044715 * y * y * y)))


def kernel(x, w_mat):
    m, k_per = x.shape
    _, n = w_mat.shape
    chunk = m // N_DEV
    cw = n // H
    bf16 = jnp.bfloat16
    f32 = jnp.float32

    def body(x_ref, w_ref, out_ref, *scratch):
        comms = scratch[0:2 * H]
        ssems = scratch[2 * H:4 * H]
        rsems = scratch[4 * H:6 * H]
        creds = scratch[6 * H:8 * H]

        my = lax.axis_index("i")

        p = lax.rem(my, 4)
        z = lax.div(my, 4)
        q = jnp.where(
            p == 0, z,
            jnp.where(p == 3, 7 - z, jnp.where(p == 2, 8 + z, 15 - z)),
        )

        def perm(r):
            return jnp.where(
                r < 4, 4 * r,
                jnp.where(r < 8, 31 - 4 * r,
                          jnp.where(r < 12, 4 * r - 30, 61 - 4 * r)),
            )

        left = perm(lax.rem(q + N_DEV - 1, N_DEV))
        right = perm(lax.rem(q + 1, N_DEV))

        def row(off):
            return lax.rem(q + off + 2 * N_DEV, N_DEV) * chunk

        def matmul(off):
            r0 = row(off)
            out_ref[pl.ds(r0, chunk), :] = jnp.dot(
                x_ref[pl.ds(r0, chunk), :], w_ref[...],
                preferred_element_type=f32,
            )

        def acc_q(off, co):
            return out_ref[pl.ds(row(off), chunk), co:co + cw]

        class Lane:
            def __init__(self, i, dst, cred_to, co, last):
                self.comm, self.ssem = comms[i], ssems[i]
                self.rsem, self.cred = rsems[i], creds[i]
                self.dst, self.cred_to, self.co, self.last = dst, cred_to, co, last

            def mk(self, k):
                return pltpu.make_async_remote_copy(
                    src_ref=self.comm.at[k % DEPTH],
                    dst_ref=self.comm.at[(k + 1) % DEPTH],
                    send_sem=self.ssem.at[k % DEPTH],
                    recv_sem=self.rsem.at[(k + 1) % DEPTH],
                    device_id=(self.dst,),
                    device_id_type=MESH,
                )

            def issue(self, k):
                if k >= DEPTH - 1:
                    pl.semaphore_wait(self.cred, 1)
                self.mk(k).start()

            def finish_send(self, k):
                self.mk(k).wait_send()
                if k <= self.last - (DEPTH - 1):
                    pl.semaphore_signal(self.cred, inc=1,
                                        device_id=(self.cred_to,),
                                        device_id_type=MESH)

        r_lanes = [Lane(j, right, left, j * cw, R_LAST) for j in range(H)]
        l_lanes = [Lane(H + j, left, right, j * cw, L_LAST) for j in range(H)]

        def add_r(lane, k):
            rs = (k + 1) % DEPTH
            lane.comm[rs, :, :] = (
                lane.comm[rs, :, :] + acc_q(7 - k, lane.co).astype(bf16)
            )

        def add_l(lane, k):
            rs = (k + 1) % DEPTH
            lane.comm[rs, :, :] = (
                lane.comm[rs, :, :] + acc_q(k - 6, lane.co).astype(bf16)
            )

        def store_r(lane, k):
            rs = (k + 1) % DEPTH
            out_ref[pl.ds(row(7 - k), chunk), lane.co:lane.co + cw] = (
                lane.comm[rs, :, :].astype(f32)
            )

        def store_l(lane, k):
            rs = (k + 1) % DEPTH
            out_ref[pl.ds(row(k - 6), chunk), lane.co:lane.co + cw] = (
                lane.comm[rs, :, :].astype(f32)
            )

        def combine(r_lane, l_lane):
            r_fin = (7 + 1) % DEPTH
            l_fin = (6 + 1) % DEPTH
            co = r_lane.co
            total = (r_lane.comm[r_fin, :, :].astype(f32)
                     + l_lane.comm[l_fin, :, :].astype(f32) + acc_q(0, co))
            g = _gelu(total)
            out_ref[pl.ds(row(0), chunk), co:co + cw] = g
            gb = g.astype(bf16)
            r_lane.comm[r_fin, :, :] = gb
            l_lane.comm[l_fin, :, :] = gb

        matmul(8)
        matmul(-7)

        barrier = pltpu.get_barrier_semaphore()
        for nbr in (left, right):
            pl.semaphore_signal(barrier, inc=1, device_id=(nbr,),
                                device_id_type=MESH)
        pl.semaphore_wait(barrier, 2)

        for lane in r_lanes:
            lane.comm[0, :, :] = acc_q(8, lane.co).astype(bf16)
        for lane in l_lanes:
            lane.comm[0, :, :] = acc_q(-7, lane.co).astype(bf16)

        pend_r = pend_l = None
        for s in range(17):
            ra, la = r_lanes[0], l_lanes[0]
            if s <= R_LAST:
                ra.issue(s)
            if 1 <= s <= L_LAST + 1:
                la.issue(s - 1)
            if s == 0:
                for off in (7, -6, 6, -5, 5, -4, 4, -3, 3, -2, 2, -1, 1, 0):
                    matmul(off)
            if pend_r is not None:
                store_r(ra, pend_r)
                pend_r = None
            if pend_l is not None:
                store_l(la, pend_l)
                pend_l = None
            for j in range(1, H):
                rj, lj = r_lanes[j], l_lanes[j]
                if s == 8:
                    rj.finish_send(7)
                    rj.mk(7).wait_recv()
                    lj.finish_send(6)
                    lj.mk(6).wait_recv()
                    combine(rj, lj)
                    rj.issue(8)
                    lj.issue(7)
                else:
                    if 1 <= s <= R_LAST + 1:
                        rj.finish_send(s - 1)
                        rj.mk(s - 1).wait_recv()
                        if s - 1 <= 6:
                            add_r(rj, s - 1)
                    if s <= R_LAST:
                        rj.issue(s)
                    if 2 <= s <= L_LAST + 2:
                        lj.finish_send(s - 2)
                        lj.mk(s - 2).wait_recv()
                        if s - 2 <= 5:
                            add_l(lj, s - 2)
                    if 1 <= s <= L_LAST + 1:
                        lj.issue(s - 1)
                if 1 <= s <= R_LAST + 1 and s - 1 >= 8:
                    store_r(rj, s - 1)
                if 2 <= s <= L_LAST + 2 and s - 2 >= 7:
                    store_l(lj, s - 2)
            if s <= R_LAST:
                ra.finish_send(s)
                ra.mk(s).wait_recv()
                if s <= 6:
                    add_r(ra, s)
                elif s >= 8:
                    pend_r = s
            if 1 <= s <= L_LAST + 1:
                la.finish_send(s - 1)
                la.mk(s - 1).wait_recv()
                if s - 1 <= 5:
                    add_l(la, s - 1)
                elif s - 1 >= 7:
                    pend_l = s - 1
            if s == 7:
                combine(ra, la)

    comm_shape = pltpu.VMEM((DEPTH, chunk, cw), bf16)
    dma = pltpu.SemaphoreType.DMA((DEPTH,))
    reg = pltpu.SemaphoreType.REGULAR
    return pl.pallas_call(
        body,
        out_shape=jax.ShapeDtypeStruct((m, n), f32),
        in_specs=[
            pl.BlockSpec(memory_space=pltpu.VMEM),
            pl.BlockSpec(memory_space=pltpu.VMEM),
        ],
        out_specs=pl.BlockSpec(memory_space=pltpu.VMEM),
        scratch_shapes=(
            [comm_shape] * (2 * H)
            + [dma] * (2 * H)
            + [dma] * (2 * H)
            + [reg] * (2 * H)
        ),
        compiler_params=pltpu.CompilerParams(collective_id=0),
    )(x, w_mat)
